# baseline (speedup 1.0000x reference)
import functools

import jax
import jax.numpy as jnp
from jax import lax
from jax.experimental import pallas as pl
from jax.experimental.pallas import tpu as pltpu
from jax.experimental.pallas import tpu_sc as plsc

_info = plsc.get_sparse_core_info()
_NC = _info.num_cores
_NS = _info.num_subcores
_NW = _NC * _NS

_CHUNK = 512


@functools.lru_cache(maxsize=None)
def _make_gather(B, V, D):
    assert B % (_NW * _CHUNK) == 0
    b_per_w = B // _NW
    n_chunks = b_per_w // _CHUNK
    mesh = plsc.VectorSubcoreMesh(core_axis_name="c", subcore_axis_name="s")

    @functools.partial(
        pl.kernel,
        mesh=mesh,
        compiler_params=pltpu.CompilerParams(use_tc_tiling_on_sc=False),
        out_type=jax.ShapeDtypeStruct((B, D), jnp.float32),
        scratch_types=[
            pltpu.VMEM((_CHUNK,), jnp.int32),
            pltpu.VMEM((_CHUNK, D), jnp.float32),
            pltpu.SemaphoreType.DMA,
        ],
    )
    def gather_kernel(idx_hbm, table_hbm, out_hbm, idx_v, rows_v, sem):
        wid = lax.axis_index("s") * _NC + lax.axis_index("c")
        base = wid * b_per_w

        def body(c, carry):
            off = base + c * _CHUNK
            pltpu.sync_copy(idx_hbm.at[pl.ds(off, _CHUNK)], idx_v)
            pltpu.async_copy(table_hbm.at[idx_v], rows_v, sem).wait()
            pltpu.sync_copy(rows_v, out_hbm.at[pl.ds(off, _CHUNK)])
            return carry

        lax.fori_loop(0, n_chunks, body, 0)

    return gather_kernel


def kernel(x, table):
    V, D = table.shape
    xf = x.reshape(-1).astype(jnp.int32)
    out = _make_gather(xf.shape[0], V, D)(xf, table)
    return out.reshape(x.shape + (D,))


# speedup vs baseline: 1.0421x; 1.0421x over previous
import functools

import jax
import jax.numpy as jnp
from jax import lax
from jax.experimental import pallas as pl
from jax.experimental.pallas import tpu as pltpu
from jax.experimental.pallas import tpu_sc as plsc

_info = plsc.get_sparse_core_info()
_NC = _info.num_cores
_NS = _info.num_subcores
_NW = _NC * _NS

_CHUNK = 512


@functools.lru_cache(maxsize=None)
def _make_gather(B, V, D):
    assert B % (_NW * 2 * _CHUNK) == 0
    b_per_w = B // _NW
    n_chunks = b_per_w // _CHUNK
    n_groups = n_chunks // 2
    mesh = plsc.VectorSubcoreMesh(core_axis_name="c", subcore_axis_name="s")

    @functools.partial(
        pl.kernel,
        mesh=mesh,
        compiler_params=pltpu.CompilerParams(use_tc_tiling_on_sc=False),
        out_type=jax.ShapeDtypeStruct((B, D), jnp.float32),
        scratch_types=[
            pltpu.VMEM((b_per_w,), jnp.int32),
            pltpu.VMEM((_CHUNK, D), jnp.float32),
            pltpu.VMEM((_CHUNK, D), jnp.float32),
            pltpu.SemaphoreType.DMA,
            pltpu.SemaphoreType.DMA,
            pltpu.SemaphoreType.DMA,
            pltpu.SemaphoreType.DMA,
        ],
    )
    def gather_kernel(idx_hbm, table_hbm, out_hbm, idx_all, rows0, rows1,
                      gsem0, gsem1, osem0, osem1):
        wid = lax.axis_index("s") * _NC + lax.axis_index("c")
        base = wid * b_per_w
        pltpu.sync_copy(idx_hbm.at[pl.ds(base, b_per_w)], idx_all)

        def gather(c, buf, sem):
            return pltpu.make_async_copy(
                table_hbm.at[idx_all.at[pl.ds(c * _CHUNK, _CHUNK)]], buf, sem)

        def wback(c, buf, sem):
            return pltpu.make_async_copy(
                buf, out_hbm.at[pl.ds(base + c * _CHUNK, _CHUNK)], sem)

        gather(0, rows0, gsem0).start()

        def body(g, carry):
            c0 = 2 * g
            c1 = c0 + 1
            gather(c0, rows0, gsem0).wait()

            @pl.when(g > 0)
            def _():
                wback(c1 - 2, rows1, osem1).wait()

            gather(c1, rows1, gsem1).start()
            wback(c0, rows0, osem0).start()
            gather(c1, rows1, gsem1).wait()
            wback(c0, rows0, osem0).wait()

            @pl.when(g < n_groups - 1)
            def _():
                gather(c0 + 2, rows0, gsem0).start()

            wback(c1, rows1, osem1).start()
            return carry

        lax.fori_loop(0, n_groups, body, 0)
        wback(n_chunks - 1, rows1, osem1).wait()

    return gather_kernel


def kernel(x, table):
    V, D = table.shape
    xf = x.reshape(-1).astype(jnp.int32)
    out = _make_gather(xf.shape[0], V, D)(xf, table)
    return out.reshape(x.shape + (D,))


# speedup vs baseline: 1.0425x; 1.0004x over previous
import functools

import jax
import jax.numpy as jnp
from jax import lax
from jax.experimental import pallas as pl
from jax.experimental.pallas import tpu as pltpu
from jax.experimental.pallas import tpu_sc as plsc

_info = plsc.get_sparse_core_info()
_NC = _info.num_cores
_NS = _info.num_subcores
_NW = _NC * _NS

_R = 4


@functools.lru_cache(maxsize=None)
def _make_gather(B, H, V, D):
    assert B % (_NW * 2 * _R) == 0
    rows_per_w = B // _NW
    n_chunks = rows_per_w // _R
    n_groups = n_chunks // 2
    mesh = plsc.VectorSubcoreMesh(core_axis_name="c", subcore_axis_name="s")

    @functools.partial(
        pl.kernel,
        mesh=mesh,
        compiler_params=pltpu.CompilerParams(use_tc_tiling_on_sc=False),
        out_type=jax.ShapeDtypeStruct((B // _R, _R * H, D), jnp.float32),
        scratch_types=[
            pltpu.VMEM((n_chunks, _R * H), jnp.int32),
            pltpu.VMEM((_R * H, D), jnp.float32),
            pltpu.VMEM((_R * H, D), jnp.float32),
            pltpu.SemaphoreType.DMA,
            pltpu.SemaphoreType.DMA,
            pltpu.SemaphoreType.DMA,
            pltpu.SemaphoreType.DMA,
        ],
    )
    def gather_kernel(x_hbm, table_hbm, out_hbm, idx_all, rows0, rows1,
                      gsem0, gsem1, osem0, osem1):
        wid = lax.axis_index("s") * _NC + lax.axis_index("c")
        cbase = wid * n_chunks
        pltpu.sync_copy(x_hbm.at[pl.ds(cbase, n_chunks)], idx_all)

        def gather(c, buf, sem):
            return pltpu.make_async_copy(
                table_hbm.at[idx_all.at[c]], buf, sem)

        def wback(c, buf, sem):
            return pltpu.make_async_copy(buf, out_hbm.at[cbase + c], sem)

        gather(0, rows0, gsem0).start()

        def body(g, carry):
            c0 = 2 * g
            c1 = c0 + 1
            gather(c0, rows0, gsem0).wait()

            @pl.when(g > 0)
            def _():
                wback(c1 - 2, rows1, osem1).wait()

            gather(c1, rows1, gsem1).start()
            wback(c0, rows0, osem0).start()
            gather(c1, rows1, gsem1).wait()
            wback(c0, rows0, osem0).wait()

            @pl.when(g < n_groups - 1)
            def _():
                gather(c0 + 2, rows0, gsem0).start()

            wback(c1, rows1, osem1).start()
            return carry

        lax.fori_loop(0, n_groups, body, 0)
        wback(n_chunks - 1, rows1, osem1).wait()

    return gather_kernel


def kernel(x, table):
    V, D = table.shape
    B, H = x.shape
    x2 = x.astype(jnp.int32).reshape(B // _R, _R * H)
    out = _make_gather(B, H, V, D)(x2, table)
    return out.reshape(B, H, D)


# speedup vs baseline: 1.0974x; 1.0527x over previous
import functools

import jax
import jax.numpy as jnp
from jax import lax
from jax.experimental import pallas as pl
from jax.experimental.pallas import tpu as pltpu
from jax.experimental.pallas import tpu_sc as plsc

_info = plsc.get_sparse_core_info()
_NC = _info.num_cores
_NS = _info.num_subcores
_NW = _NC * _NS

_R = 4


@functools.lru_cache(maxsize=None)
def _make_gather(B, H, V2, D):
    assert B % (_NW * 2 * _R) == 0
    rows_per_w = B // _NW
    n_chunks = rows_per_w // _R
    n_groups = n_chunks // 2
    mesh = plsc.VectorSubcoreMesh(core_axis_name="c", subcore_axis_name="s")

    @functools.partial(
        pl.kernel,
        mesh=mesh,
        compiler_params=pltpu.CompilerParams(use_tc_tiling_on_sc=False),
        out_type=jax.ShapeDtypeStruct((B, H, D), jnp.float32),
        scratch_types=[
            pltpu.VMEM((n_chunks, _R * H), jnp.int32),
            pltpu.VMEM((_R * H, D), jnp.float32),
            pltpu.VMEM((_R * H, D), jnp.float32),
            pltpu.SemaphoreType.DMA,
            pltpu.SemaphoreType.DMA,
            pltpu.SemaphoreType.DMA,
            pltpu.SemaphoreType.DMA,
        ],
    )
    def gather_kernel(x_hbm, table_hbm, out_hbm, idx_all, rows0, rows1,
                      gsem0, gsem1, osem0, osem1):
        wid = lax.axis_index("s") * _NC + lax.axis_index("c")
        cbase = wid * n_chunks
        pltpu.sync_copy(x_hbm.at[pl.ds(cbase, n_chunks)], idx_all)

        def gather(c, buf, sem):
            return pltpu.make_async_copy(
                table_hbm.at[idx_all.at[c]], buf, sem)

        def wback_start(c, buf, sem):
            b0 = (cbase + c) * _R
            for j in range(_R):
                pltpu.make_async_copy(
                    buf.at[pl.ds(j * H, H)], out_hbm.at[b0 + j], sem).start()

        def wback_wait(c, buf, sem):
            b0 = (cbase + c) * _R
            for j in range(_R):
                pltpu.make_async_copy(
                    buf.at[pl.ds(j * H, H)], out_hbm.at[b0 + j], sem).wait()

        gather(0, rows0, gsem0).start()

        def body(g, carry):
            c0 = 2 * g
            c1 = c0 + 1
            gather(c0, rows0, gsem0).wait()

            @pl.when(g > 0)
            def _():
                wback_wait(c1 - 2, rows1, osem1)

            gather(c1, rows1, gsem1).start()
            wback_start(c0, rows0, osem0)
            gather(c1, rows1, gsem1).wait()
            wback_wait(c0, rows0, osem0)

            @pl.when(g < n_groups - 1)
            def _():
                gather(c0 + 2, rows0, gsem0).start()

            wback_start(c1, rows1, osem1)
            return carry

        lax.fori_loop(0, n_groups, body, 0)
        wback_wait(n_chunks - 1, rows1, osem1)

    return gather_kernel


def kernel(x, table):
    V, D = table.shape
    B, H = x.shape
    tp = jnp.pad(table, ((0, 0), (0, 128 - D))).reshape(2 * V, D)
    x2 = (x.astype(jnp.int32) * 2).reshape(B // _R, _R * H)
    return _make_gather(B, H, 2 * V, D)(x2, tp)


# speedup vs baseline: 1.2731x; 1.1600x over previous
import functools

import jax
import jax.numpy as jnp
from jax import lax
from jax.experimental import pallas as pl
from jax.experimental.pallas import tpu as pltpu
from jax.experimental.pallas import tpu_sc as plsc

_info = plsc.get_sparse_core_info()
_NC = _info.num_cores
_NS = _info.num_subcores
_NW = _NC * _NS

_R = 2


@functools.lru_cache(maxsize=None)
def _make_gather(B, H, V, D):
    assert B % (_NW * 2 * _R) == 0
    rows_per_w = B // _NW
    n_idx_w = rows_per_w * H
    n_chunks = rows_per_w // _R
    n_groups = n_chunks // 2
    mesh = plsc.VectorSubcoreMesh(core_axis_name="c", subcore_axis_name="s")

    @functools.partial(
        pl.kernel,
        mesh=mesh,
        compiler_params=pltpu.CompilerParams(use_tc_tiling_on_sc=True),
        out_type=jax.ShapeDtypeStruct((B * H, 2 * D), jnp.float32),
        scratch_types=[
            pltpu.VMEM((n_idx_w,), jnp.int32),
            pltpu.VMEM((_R * H, 2 * D), jnp.float32),
            pltpu.VMEM((_R * H, 2 * D), jnp.float32),
            pltpu.SemaphoreType.DMA,
            pltpu.SemaphoreType.DMA,
            pltpu.SemaphoreType.DMA,
            pltpu.SemaphoreType.DMA,
        ],
    )
    def gather_kernel(x_hbm, table_hbm, out_hbm, idx_all, rows0, rows1,
                      gsem0, gsem1, osem0, osem1):
        wid = lax.axis_index("s") * _NC + lax.axis_index("c")
        base = wid * rows_per_w
        pltpu.sync_copy(x_hbm.at[pl.ds(base * H, n_idx_w)], idx_all)

        def gather(c, buf, sem):
            return pltpu.make_async_copy(
                table_hbm.at[idx_all.at[pl.ds(c * _R * H, _R * H)]], buf, sem)

        def wback(c, buf, sem):
            r0 = (base + c * _R) * H
            return pltpu.make_async_copy(
                buf, out_hbm.at[pl.ds(r0, _R * H)], sem)

        gather(0, rows0, gsem0).start()

        def body(g, carry):
            c0 = 2 * g
            c1 = c0 + 1
            gather(c0, rows0, gsem0).wait()

            @pl.when(g > 0)
            def _():
                wback(c1 - 2, rows1, osem1).wait()

            gather(c1, rows1, gsem1).start()
            wback(c0, rows0, osem0).start()
            gather(c1, rows1, gsem1).wait()
            wback(c0, rows0, osem0).wait()

            @pl.when(g < n_groups - 1)
            def _():
                gather(c0 + 2, rows0, gsem0).start()

            wback(c1, rows1, osem1).start()
            return carry

        lax.fori_loop(0, n_groups, body, 0)
        wback(n_chunks - 1, rows1, osem1).wait()

    return gather_kernel


def kernel(x, table):
    V, D = table.shape
    B, H = x.shape
    tp = jnp.pad(table, ((0, 0), (0, 128 - D)))
    x2 = x.astype(jnp.int32).reshape(B * H)
    out = _make_gather(B, H, V, D)(x2, tp)
    return out.reshape(B, H, 2 * D)[:, :, :D]


# speedup vs baseline: 1.4850x; 1.1665x over previous
import functools

import jax
import jax.numpy as jnp
from jax import lax
from jax.experimental import pallas as pl
from jax.experimental.pallas import tpu as pltpu
from jax.experimental.pallas import tpu_sc as plsc

_info = plsc.get_sparse_core_info()
_NC = _info.num_cores
_NS = _info.num_subcores
_NW = _NC * _NS

_R = 2


@functools.lru_cache(maxsize=None)
def _make_gather(B, H, V, D):
    assert B % (_NW * 2 * _R) == 0
    rows_per_w = B // _NW
    n_idx_w = rows_per_w * H
    n_chunks = rows_per_w // _R
    n_groups = n_chunks // 2
    mesh = plsc.VectorSubcoreMesh(core_axis_name="c", subcore_axis_name="s")

    @functools.partial(
        pl.kernel,
        mesh=mesh,
        compiler_params=pltpu.CompilerParams(use_tc_tiling_on_sc=False),
        out_type=jax.ShapeDtypeStruct((B * H, 2 * D), jnp.float32),
        scratch_types=[
            pltpu.VMEM((n_idx_w,), jnp.int32),
            pltpu.VMEM((_R * H, D), jnp.float32),
            pltpu.VMEM((_R * H, D), jnp.float32),
            pltpu.SemaphoreType.DMA,
            pltpu.SemaphoreType.DMA,
            pltpu.SemaphoreType.DMA,
            pltpu.SemaphoreType.DMA,
        ],
    )
    def gather_kernel(x_hbm, table_hbm, out_hbm, idx_all, rows0, rows1,
                      gsem0, gsem1, osem0, osem1):
        wid = lax.axis_index("s") * _NC + lax.axis_index("c")
        base = wid * rows_per_w
        pltpu.sync_copy(x_hbm.at[pl.ds(base * H, n_idx_w)], idx_all)

        def gather(c, buf, sem):
            return pltpu.make_async_copy(
                table_hbm.at[idx_all.at[pl.ds(c * _R * H, _R * H)]], buf, sem)

        def wback(c, buf, sem):
            r0 = (base + c * _R) * H
            return pltpu.make_async_copy(
                buf, out_hbm.at[pl.ds(r0, _R * H), pl.ds(0, D)], sem)

        gather(0, rows0, gsem0).start()

        def body(g, carry):
            c0 = 2 * g
            c1 = c0 + 1
            gather(c0, rows0, gsem0).wait()

            @pl.when(g > 0)
            def _():
                wback(c1 - 2, rows1, osem1).wait()

            gather(c1, rows1, gsem1).start()
            wback(c0, rows0, osem0).start()
            gather(c1, rows1, gsem1).wait()
            wback(c0, rows0, osem0).wait()

            @pl.when(g < n_groups - 1)
            def _():
                gather(c0 + 2, rows0, gsem0).start()

            wback(c1, rows1, osem1).start()
            return carry

        lax.fori_loop(0, n_groups, body, 0)
        wback(n_chunks - 1, rows1, osem1).wait()

    return gather_kernel


def kernel(x, table):
    V, D = table.shape
    B, H = x.shape
    tp = jnp.pad(table, ((0, 0), (0, 128 - D))).reshape(2 * V, D)
    x2 = (x.astype(jnp.int32) * 2).reshape(B * H)
    out = _make_gather(B, H, V, D)(x2, tp)
    return out.reshape(B, H, 2 * D)[:, :, :D]
